# Initial kernel scaffold; baseline (speedup 1.0000x reference)
import jax
import jax.numpy as jnp
from jax.experimental import pallas as pl


def kernel(logits, val_freqs):
    raise NotImplementedError("write your pallas kernel here")


# speedup vs baseline: 1922.1505x; 1922.1505x over previous
import functools

import jax
import jax.numpy as jnp
from jax.experimental import pallas as pl
from jax.experimental.pallas import tpu as pltpu

NB = 15
C = 19
BH = 128


def _body(vf_ref, x_ref, o_ref):
    x = x_ref[...]
    m = jnp.max(x, axis=1, keepdims=True)
    e = jnp.exp(x - m)
    s = jnp.sum(e, axis=1, keepdims=True)
    q = (e / s) * (1.0 / (1.0 / NB))
    cols = []
    for c in range(C):
        qc = q[:, c]
        cc = jnp.full(qc.shape, vf_ref[c, 0], dtype=jnp.float32)
        for k in range(1, NB):
            cc = jnp.where(qc >= float(k), vf_ref[c, k], cc)
        cols.append(cc[:, None])
    cal = jnp.concatenate(cols, axis=1)
    s2 = jnp.sum(cal, axis=1, keepdims=True)
    s2 = jnp.where(s2 == 0.0, 1.0, s2)
    o_ref[...] = cal / s2


def kernel(logits, val_freqs):
    B, c, H, W = logits.shape
    grid = (B, H // BH)
    return pl.pallas_call(
        _body,
        grid=grid,
        in_specs=[
            pl.BlockSpec(memory_space=pltpu.SMEM),
            pl.BlockSpec((1, c, BH, W), lambda b, h: (b, 0, h, 0)),
        ],
        out_specs=pl.BlockSpec((1, c, BH, W), lambda b, h: (b, 0, h, 0)),
        out_shape=jax.ShapeDtypeStruct(logits.shape, jnp.float32),
    )(val_freqs, logits)
